# baseline (speedup 1.0000x reference)
import functools

import jax
import jax.numpy as jnp
from jax import lax
from jax.experimental import pallas as pl
from jax.experimental.pallas import tpu as pltpu
from jax.experimental.pallas import tpu_sc as plsc

POP = 1000000
DELAY = 8
LANE = 128
NTILES = 7812
TAIL = POP - NTILES * LANE
TA = 488
TB = 489
HA = 244 * LANE
HB1 = 244 * LANE
HB2 = 245 * LANE

_mesh = plsc.VectorSubcoreMesh(
    core_axis_name="c", subcore_axis_name="s", num_cores=1)


@functools.partial(
    pl.kernel,
    mesh=_mesh,
    out_type=jax.ShapeDtypeStruct((POP,), jnp.float32),
    scratch_types=[
        pltpu.VMEM((16,), jnp.int32),
        pltpu.VMEM((1, HB2), jnp.float32),
        pltpu.VMEM((1, HB2), jnp.float32),
        pltpu.VMEM((8, TAIL), jnp.float32),
        pltpu.SemaphoreType.DMA,
        pltpu.SemaphoreType.DMA,
        pltpu.SemaphoreType.DMA,
        pltpu.SemaphoreType.DMA,
    ],
)
def _delayed_row_copy(hist_hbm, out_hbm, idx_v, row_a, row_b, tail_v,
                      sem_ga, sem_gb, sem_oa, sem_ob):
    wid = lax.axis_index("s")
    idx_v[...] = jnp.full((16,), DELAY - 1, jnp.int32)
    idx1 = idx_v.at[pl.ds(0, 1)]

    def two_half_copy(base, h1, h2):
        ga = pltpu.async_copy(
            hist_hbm.at[idx1, pl.ds(base, h1)],
            row_a.at[pl.ds(0, 1), pl.ds(0, h1)], sem_ga)
        gb = pltpu.async_copy(
            hist_hbm.at[idx1, pl.ds(base + h1, h2)],
            row_b.at[pl.ds(0, 1), pl.ds(0, h2)], sem_gb)
        return ga, gb

    def two_half_drain(base, h1, h2, ga, gb):
        ga.wait()
        oa = pltpu.async_copy(
            row_a.at[0, pl.ds(0, h1)], out_hbm.at[pl.ds(base, h1)], sem_oa)
        gb.wait()
        ob = pltpu.async_copy(
            row_b.at[0, pl.ds(0, h2)],
            out_hbm.at[pl.ds(base + h1, h2)], sem_ob)
        oa.wait()
        ob.wait()

    @pl.when(wid < 12)
    def _():
        base = wid * (TA * LANE)
        ga, gb = two_half_copy(base, HA, HA)

        @pl.when(wid == 0)
        def _():
            tbase = NTILES * LANE
            pltpu.sync_copy(
                hist_hbm.at[pl.ds(0, 8), pl.ds(tbase, TAIL)], tail_v)
            pltpu.sync_copy(
                tail_v.at[DELAY - 1], out_hbm.at[pl.ds(tbase, TAIL)])

        two_half_drain(base, HA, HA, ga, gb)

    @pl.when(wid >= 12)
    def _():
        base = (TA * wid + (wid - 12)) * LANE
        ga, gb = two_half_copy(base, HB1, HB2)
        two_half_drain(base, HB1, HB2, ga, gb)


def kernel(s, spike_history):
    return _delayed_row_copy(spike_history)


# speedup vs baseline: 1.0208x; 1.0208x over previous
import functools

import jax
import jax.numpy as jnp
from jax import lax
from jax.experimental import pallas as pl
from jax.experimental.pallas import tpu as pltpu
from jax.experimental.pallas import tpu_sc as plsc

POP = 1000000
DELAY = 8
NWORK = 31
CHUNK = 252 * 128
NSUB = 4
SUB = CHUNK // NSUB
TAIL = POP - NWORK * CHUNK

_mesh = plsc.VectorSubcoreMesh(core_axis_name="c", subcore_axis_name="s")


@functools.partial(
    pl.kernel,
    mesh=_mesh,
    out_type=jax.ShapeDtypeStruct((POP,), jnp.float32),
    scratch_types=[
        pltpu.VMEM((16,), jnp.int32),
        pltpu.VMEM((NSUB, 1, SUB), jnp.float32),
        pltpu.SemaphoreType.DMA,
        pltpu.SemaphoreType.DMA,
        pltpu.SemaphoreType.DMA,
        pltpu.SemaphoreType.DMA,
        pltpu.SemaphoreType.DMA,
        pltpu.SemaphoreType.DMA,
        pltpu.SemaphoreType.DMA,
        pltpu.SemaphoreType.DMA,
        pltpu.VMEM((8, TAIL), jnp.float32),
    ],
)
def _delayed_row_copy(hist_hbm, out_hbm, idx_v, rows_v,
                      g0, g1, g2, g3, o0, o1, o2, o3, tail_v):
    wid = lax.axis_index("s") * 2 + lax.axis_index("c")
    idx_v[...] = jnp.full((16,), DELAY - 1, jnp.int32)
    idx1 = idx_v.at[pl.ds(0, 1)]
    gsems = (g0, g1, g2, g3)
    osems = (o0, o1, o2, o3)

    @pl.when(wid < NWORK)
    def _():
        base = wid * CHUNK

        def gather(k):
            return pltpu.async_copy(
                hist_hbm.at[idx1, pl.ds(base + k * SUB, SUB)],
                rows_v.at[k], gsems[k])

        def writeout(k):
            return pltpu.async_copy(
                rows_v.at[k, 0], out_hbm.at[pl.ds(base + k * SUB, SUB)],
                osems[k])

        gathers = [gather(0), gather(1)]
        writes = []
        for k in range(NSUB):
            gathers[k].wait()
            writes.append(writeout(k))
            if k + 2 < NSUB:
                gathers.append(gather(k + 2))
        for w in writes:
            w.wait()

    @pl.when(wid == NWORK)
    def _():
        base = NWORK * CHUNK
        pltpu.sync_copy(hist_hbm.at[pl.ds(0, 8), pl.ds(base, TAIL)], tail_v)
        pltpu.sync_copy(tail_v.at[DELAY - 1], out_hbm.at[pl.ds(base, TAIL)])


def kernel(s, spike_history):
    return _delayed_row_copy(spike_history)


# speedup vs baseline: 1.0495x; 1.0281x over previous
import functools

import jax
import jax.numpy as jnp
from jax import lax
from jax.experimental import pallas as pl
from jax.experimental.pallas import tpu as pltpu
from jax.experimental.pallas import tpu_sc as plsc

POP = 1000000
DELAY = 8
NWORK = 31
CHUNK = 252 * 128
HALF = CHUNK // 2
TAIL = POP - NWORK * CHUNK

_mesh = plsc.VectorSubcoreMesh(core_axis_name="c", subcore_axis_name="s")


@functools.partial(
    pl.kernel,
    mesh=_mesh,
    out_type=jax.ShapeDtypeStruct((POP,), jnp.float32),
    scratch_types=[
        pltpu.VMEM((16,), jnp.int32),
        pltpu.VMEM((1, HALF), jnp.float32),
        pltpu.VMEM((1, HALF), jnp.float32),
        pltpu.VMEM((8, TAIL), jnp.float32),
        pltpu.SemaphoreType.DMA,
        pltpu.SemaphoreType.DMA,
        pltpu.SemaphoreType.DMA,
        pltpu.SemaphoreType.DMA,
    ],
)
def _delayed_row_copy(hist_hbm, out_hbm, idx_v, row_a, row_b, tail_v,
                      sem_ga, sem_gb, sem_oa, sem_ob):
    wid = lax.axis_index("s") * 2 + lax.axis_index("c")
    idx_v[...] = jnp.full((16,), DELAY - 1, jnp.int32)
    idx1 = idx_v.at[pl.ds(0, 1)]

    @pl.when(wid < NWORK)
    def _():
        base = wid * CHUNK
        ga = pltpu.async_copy(
            hist_hbm.at[idx1, pl.ds(base, HALF)], row_a, sem_ga)
        gb = pltpu.async_copy(
            hist_hbm.at[idx1, pl.ds(base + HALF, HALF)], row_b, sem_gb)
        ga.wait()
        oa = pltpu.async_copy(
            row_a.at[0], out_hbm.at[pl.ds(base, HALF)], sem_oa)
        gb.wait()
        ob = pltpu.async_copy(
            row_b.at[0], out_hbm.at[pl.ds(base + HALF, HALF)], sem_ob)
        oa.wait()
        ob.wait()

    @pl.when(wid == NWORK)
    def _():
        base = NWORK * CHUNK
        pltpu.sync_copy(hist_hbm.at[pl.ds(0, 8), pl.ds(base, TAIL)], tail_v)
        pltpu.sync_copy(tail_v.at[DELAY - 1], out_hbm.at[pl.ds(base, TAIL)])


def kernel(s, spike_history):
    return _delayed_row_copy(spike_history)
